# Initial kernel scaffold; baseline (speedup 1.0000x reference)
import jax
import jax.numpy as jnp
from jax.experimental import pallas as pl


def kernel(day, time, dow, weekday, location, W_day, W_time, W_dow, W_weekday, W_loc, ln_gamma, ln_beta):
    raise NotImplementedError("write your pallas kernel here")


# speedup vs baseline: 2.6467x; 2.6467x over previous
import functools

import jax
import jax.numpy as jnp
from jax import lax
from jax.experimental import pallas as pl
from jax.experimental.pallas import tpu as pltpu
from jax.experimental.pallas import tpu_sc as plsc

B, T = 4096, 50
NTOK = B * T
D_LOC = 64
FINAL = 192

NW = 32
BPW = NTOK // NW
CHUNK = 1280
NCHUNK = BPW // CHUNK
SUB = 128
NSUB = CHUNK // SUB

TB = 2048
GRID = NTOK // TB


def _sc_gather(table, idx):
    mesh = plsc.VectorSubcoreMesh(core_axis_name="c", subcore_axis_name="s")

    @functools.partial(
        pl.kernel,
        mesh=mesh,
        out_type=jax.ShapeDtypeStruct((NTOK, D_LOC), jnp.float32),
        compiler_params=pltpu.CompilerParams(use_tc_tiling_on_sc=False),
        scratch_types=[
            pltpu.VMEM((CHUNK,), jnp.int32),
            pltpu.VMEM((CHUNK, D_LOC), jnp.float32),
            pltpu.SemaphoreType.DMA,
        ],
    )
    def gather_kernel(table_hbm, idx_hbm, out_hbm, idx_v, rows_v, sem):
        wid = lax.axis_index("s") * 2 + lax.axis_index("c")
        base0 = wid * BPW
        for j in range(NCHUNK):
            base = base0 + j * CHUNK
            pltpu.sync_copy(idx_hbm.at[pl.ds(base, CHUNK)], idx_v)
            copies = [
                pltpu.async_copy(
                    table_hbm.at[idx_v.at[pl.ds(k * SUB, SUB)]],
                    rows_v.at[pl.ds(k * SUB, SUB)],
                    sem,
                )
                for k in range(NSUB)
            ]
            for c in copies:
                c.wait()
            pltpu.sync_copy(rows_v, out_hbm.at[pl.ds(base, CHUNK)])

    return gather_kernel(table, idx)


def _tc_body(idx_ref, loc_ref, wc_ref, g_ref, b_ref, out_ref):
    idx = idx_ref[0]
    d = idx[:, 0:1]
    tt = idx[:, 1:2] + 75
    dw = idx[:, 2:3] + 123
    wd = idx[:, 3:4] + 130
    iot = lax.broadcasted_iota(jnp.int32, (TB, 256), 1)
    oh = ((iot == d) | (iot == tt) | (iot == dw) | (iot == wd)).astype(
        jnp.float32)
    small = jnp.dot(oh, wc_ref[...], preferred_element_type=jnp.float32)
    h = jnp.concatenate([small, loc_ref[...]], axis=1)
    mean = jnp.mean(h, axis=1, keepdims=True)
    hc = h - mean
    var = jnp.mean(hc * hc, axis=1, keepdims=True)
    inv = lax.rsqrt(var + 1e-5)
    out_ref[...] = hc * inv * g_ref[...] + b_ref[...]


def kernel(day, time, dow, weekday, location, W_day, W_time, W_dow,
           W_weekday, W_loc, ln_gamma, ln_beta):
    loc_idx = location.reshape(NTOK).astype(jnp.int32)
    loc_rows = _sc_gather(W_loc, loc_idx)

    idx4 = jnp.stack(
        [day.reshape(NTOK), time.reshape(NTOK), dow.reshape(NTOK),
         weekday.reshape(NTOK)], axis=-1,
    ).astype(jnp.int32).reshape(GRID, TB, 4)

    wc = jnp.zeros((256, 128), jnp.float32)
    wc = wc.at[0:75, 0:32].set(W_day)
    wc = wc.at[75:123, 32:64].set(W_time)
    wc = wc.at[123:130, 64:96].set(W_dow)
    wc = wc.at[130:132, 96:128].set(W_weekday)

    out = pl.pallas_call(
        _tc_body,
        grid=(GRID,),
        in_specs=[
            pl.BlockSpec((1, TB, 4), lambda i: (i, 0, 0)),
            pl.BlockSpec((TB, D_LOC), lambda i: (i, 0)),
            pl.BlockSpec((256, 128), lambda i: (0, 0)),
            pl.BlockSpec((1, FINAL), lambda i: (0, 0)),
            pl.BlockSpec((1, FINAL), lambda i: (0, 0)),
        ],
        out_specs=pl.BlockSpec((TB, FINAL), lambda i: (i, 0)),
        out_shape=jax.ShapeDtypeStruct((NTOK, FINAL), jnp.float32),
    )(idx4, loc_rows, wc, ln_gamma.reshape(1, FINAL),
      ln_beta.reshape(1, FINAL))
    return out.reshape(B, T, FINAL)


# speedup vs baseline: 5.0224x; 1.8976x over previous
import functools

import jax
import jax.numpy as jnp
from jax import lax
from jax.experimental import pallas as pl
from jax.experimental.pallas import tpu as pltpu
from jax.experimental.pallas import tpu_sc as plsc

B, T = 4096, 50
NTOK = B * T
FINAL = 192

NW = 32
BPW = NTOK // NW
CHUNK = 640
NCHUNK = BPW // CHUNK
SUB = 128
NSUB = CHUNK // SUB

BB = 1024
NB = B // BB


def _sc_gather_rows(table2, idx):
    mesh = plsc.VectorSubcoreMesh(core_axis_name="c", subcore_axis_name="s")

    @functools.partial(
        pl.kernel,
        mesh=mesh,
        out_type=jax.ShapeDtypeStruct((NTOK, 128), jnp.float32),
        scratch_types=[
            pltpu.VMEM((CHUNK,), jnp.int32),
            pltpu.VMEM((CHUNK, 128), jnp.float32),
            pltpu.SemaphoreType.DMA,
        ],
    )
    def gather_kernel(table_hbm, idx_hbm, out_hbm, idx_v, rows_v, sem):
        wid = lax.axis_index("s") * 2 + lax.axis_index("c")
        base0 = wid * BPW
        for j in range(NCHUNK):
            base = base0 + j * CHUNK
            pltpu.sync_copy(idx_hbm.at[pl.ds(base, CHUNK)], idx_v)
            copies = [
                pltpu.async_copy(
                    table_hbm.at[idx_v.at[pl.ds(k * SUB, SUB)]],
                    rows_v.at[pl.ds(k * SUB, SUB)],
                    sem,
                )
                for k in range(NSUB)
            ]
            for c in copies:
                c.wait()
            pltpu.sync_copy(rows_v, out_hbm.at[pl.ds(base, CHUNK)])

    return gather_kernel(table2, idx)


def _tc_body(day_ref, time_ref, dow_ref, wd_ref, locidx_ref, loc3_ref,
             wct_ref, g_ref, b_ref, out_ref):
    t = pl.program_id(1)
    d = day_ref[pl.ds(t, 1), :]
    tt = time_ref[pl.ds(t, 1), :] + 75
    dw = dow_ref[pl.ds(t, 1), :] + 123
    wd = wd_ref[pl.ds(t, 1), :] + 130
    iot = lax.broadcasted_iota(jnp.int32, (256, BB), 0)
    oh = ((iot == d) | (iot == tt) | (iot == dw) | (iot == wd)).astype(
        jnp.float32)
    small = jnp.dot(wct_ref[...], oh,
                    preferred_element_type=jnp.float32)

    eye = (lax.broadcasted_iota(jnp.int32, (64, 128), 0) ==
           lax.broadcasted_iota(jnp.int32, (64, 128), 1)).astype(
        jnp.float32)
    locrows = loc3_ref[0]
    locsel = lax.dot_general(eye, locrows, (((1,), (1,)), ((), ())),
                             preferred_element_type=jnp.float32)

    h = jnp.concatenate([small, locsel], axis=0)
    mean = jnp.mean(h, axis=0, keepdims=True)
    hc = h - mean
    var = jnp.mean(hc * hc, axis=0, keepdims=True)
    inv = lax.rsqrt(var + 1e-5)
    out_ref[0] = hc * inv * g_ref[...] + b_ref[...]


def kernel(day, time, dow, weekday, location, W_day, W_time, W_dow,
           W_weekday, W_loc, ln_gamma, ln_beta):
    table2 = jnp.pad(W_loc, ((0, 0), (0, 64)))

    day_t = lax.transpose(day.astype(jnp.int32), (1, 0))
    time_t = lax.transpose(time.astype(jnp.int32), (1, 0))
    dow_t = lax.transpose(dow.astype(jnp.int32), (1, 0))
    wd_t = lax.transpose(weekday.astype(jnp.int32), (1, 0))
    loc_t = lax.transpose(location.astype(jnp.int32), (1, 0))

    loc_flat = loc_t.reshape(NTOK)
    loc3 = _sc_gather_rows(table2, loc_flat).reshape(T, B, 128)

    wct = jnp.zeros((128, 256), jnp.float32)
    wct = wct.at[0:32, 0:75].set(W_day.T)
    wct = wct.at[32:64, 75:123].set(W_time.T)
    wct = wct.at[64:96, 123:130].set(W_dow.T)
    wct = wct.at[96:128, 130:132].set(W_weekday.T)

    out_t = pl.pallas_call(
        _tc_body,
        grid=(NB, T),
        in_specs=[
            pl.BlockSpec((T, BB), lambda i, t: (0, i)),
            pl.BlockSpec((T, BB), lambda i, t: (0, i)),
            pl.BlockSpec((T, BB), lambda i, t: (0, i)),
            pl.BlockSpec((T, BB), lambda i, t: (0, i)),
            pl.BlockSpec((T, BB), lambda i, t: (0, i)),
            pl.BlockSpec((1, BB, 128), lambda i, t: (t, i, 0)),
            pl.BlockSpec((128, 256), lambda i, t: (0, 0)),
            pl.BlockSpec((FINAL, 1), lambda i, t: (0, 0)),
            pl.BlockSpec((FINAL, 1), lambda i, t: (0, 0)),
        ],
        out_specs=pl.BlockSpec((1, FINAL, BB), lambda i, t: (t, 0, i)),
        out_shape=jax.ShapeDtypeStruct((T, FINAL, B), jnp.float32),
    )(day_t, time_t, dow_t, wd_t, loc_t, loc3, wct,
      ln_gamma.reshape(FINAL, 1), ln_beta.reshape(FINAL, 1))

    return lax.transpose(out_t, (2, 0, 1))


# speedup vs baseline: 6.3882x; 1.2719x over previous
import functools

import jax
import jax.numpy as jnp
from jax import lax
from jax.experimental import pallas as pl
from jax.experimental.pallas import tpu as pltpu
from jax.experimental.pallas import tpu_sc as plsc

B, T = 4096, 50
NTOK = B * T
FINAL = 192

NW = 32
BPW = NTOK // NW
CHUNK = 640
NCHUNK = BPW // CHUNK
SUB = 128
NSUB = CHUNK // SUB

BB = 1024
NB = B // BB


def _sc_gather_rows(table2, idx):
    mesh = plsc.VectorSubcoreMesh(core_axis_name="c", subcore_axis_name="s")

    @functools.partial(
        pl.kernel,
        mesh=mesh,
        out_type=jax.ShapeDtypeStruct((NTOK, 128), jnp.float32),
        scratch_types=[
            pltpu.VMEM((CHUNK,), jnp.int32),
            pltpu.VMEM((CHUNK, 128), jnp.float32),
            pltpu.SemaphoreType.DMA,
        ],
    )
    def gather_kernel(table_hbm, idx_hbm, out_hbm, idx_v, rows_v, sem):
        wid = lax.axis_index("s") * 2 + lax.axis_index("c")
        base0 = wid * BPW
        for j in range(NCHUNK):
            base = base0 + j * CHUNK
            pltpu.sync_copy(idx_hbm.at[pl.ds(base, CHUNK)], idx_v)
            copies = [
                pltpu.async_copy(
                    table_hbm.at[idx_v.at[pl.ds(k * SUB, SUB)]],
                    rows_v.at[pl.ds(k * SUB, SUB)],
                    sem,
                )
                for k in range(NSUB)
            ]
            for c in copies:
                c.wait()
            pltpu.sync_copy(rows_v, out_hbm.at[pl.ds(base, CHUNK)])

    return gather_kernel(table2, idx)


NLOC = 1000000
TBLK = 4096
TGRID = (NLOC + TBLK - 1) // TBLK


def _tpad_body(wt_ref, out_ref):
    eye = (lax.broadcasted_iota(jnp.int32, (64, 128), 0) ==
           lax.broadcasted_iota(jnp.int32, (64, 128), 1)).astype(jnp.float32)
    out_ref[...] = lax.dot_general(
        wt_ref[...], eye, (((0,), (0,)), ((), ())),
        preferred_element_type=jnp.float32)


def _transpose_pad(W_loc):
    wt = lax.transpose(W_loc, (1, 0))
    return pl.pallas_call(
        _tpad_body,
        grid=(TGRID,),
        in_specs=[pl.BlockSpec((64, TBLK), lambda i: (0, i))],
        out_specs=pl.BlockSpec((TBLK, 128), lambda i: (i, 0)),
        out_shape=jax.ShapeDtypeStruct((NLOC, 128), jnp.float32),
    )(wt)


def _tc_body(day_ref, time_ref, dow_ref, wd_ref, locidx_ref, loc3_ref,
             wct_ref, g_ref, b_ref, out_ref):
    t = pl.program_id(1)
    d = day_ref[pl.ds(t, 1), :]
    tt = time_ref[pl.ds(t, 1), :] + 75
    dw = dow_ref[pl.ds(t, 1), :] + 123
    wd = wd_ref[pl.ds(t, 1), :] + 130
    iot = lax.broadcasted_iota(jnp.int32, (256, BB), 0)
    oh = ((iot == d) | (iot == tt) | (iot == dw) | (iot == wd)).astype(
        jnp.float32)
    small = jnp.dot(wct_ref[...], oh,
                    preferred_element_type=jnp.float32)

    eye = (lax.broadcasted_iota(jnp.int32, (64, 128), 0) ==
           lax.broadcasted_iota(jnp.int32, (64, 128), 1)).astype(
        jnp.float32)
    locrows = loc3_ref[0]
    locsel = lax.dot_general(eye, locrows, (((1,), (1,)), ((), ())),
                             preferred_element_type=jnp.float32)

    h = jnp.concatenate([small, locsel], axis=0)
    mean = jnp.mean(h, axis=0, keepdims=True)
    hc = h - mean
    var = jnp.mean(hc * hc, axis=0, keepdims=True)
    inv = lax.rsqrt(var + 1e-5)
    out_ref[0] = hc * inv * g_ref[...] + b_ref[...]


def kernel(day, time, dow, weekday, location, W_day, W_time, W_dow,
           W_weekday, W_loc, ln_gamma, ln_beta):
    table2 = _transpose_pad(W_loc)

    day_t = lax.transpose(day.astype(jnp.int32), (1, 0))
    time_t = lax.transpose(time.astype(jnp.int32), (1, 0))
    dow_t = lax.transpose(dow.astype(jnp.int32), (1, 0))
    wd_t = lax.transpose(weekday.astype(jnp.int32), (1, 0))
    loc_t = lax.transpose(location.astype(jnp.int32), (1, 0))

    loc_flat = loc_t.reshape(NTOK)
    loc3 = _sc_gather_rows(table2, loc_flat).reshape(T, B, 128)

    wct = jnp.zeros((128, 256), jnp.float32)
    wct = wct.at[0:32, 0:75].set(W_day.T)
    wct = wct.at[32:64, 75:123].set(W_time.T)
    wct = wct.at[64:96, 123:130].set(W_dow.T)
    wct = wct.at[96:128, 130:132].set(W_weekday.T)

    out_t = pl.pallas_call(
        _tc_body,
        grid=(NB, T),
        in_specs=[
            pl.BlockSpec((T, BB), lambda i, t: (0, i)),
            pl.BlockSpec((T, BB), lambda i, t: (0, i)),
            pl.BlockSpec((T, BB), lambda i, t: (0, i)),
            pl.BlockSpec((T, BB), lambda i, t: (0, i)),
            pl.BlockSpec((T, BB), lambda i, t: (0, i)),
            pl.BlockSpec((1, BB, 128), lambda i, t: (t, i, 0)),
            pl.BlockSpec((128, 256), lambda i, t: (0, 0)),
            pl.BlockSpec((FINAL, 1), lambda i, t: (0, 0)),
            pl.BlockSpec((FINAL, 1), lambda i, t: (0, 0)),
        ],
        out_specs=pl.BlockSpec((1, FINAL, BB), lambda i, t: (t, 0, i)),
        out_shape=jax.ShapeDtypeStruct((T, FINAL, B), jnp.float32),
    )(day_t, time_t, dow_t, wd_t, loc_t, loc3, wct,
      ln_gamma.reshape(FINAL, 1), ln_beta.reshape(FINAL, 1))

    return lax.transpose(out_t, (2, 0, 1))


# speedup vs baseline: 6.5698x; 1.0284x over previous
import functools

import jax
import jax.numpy as jnp
from jax import lax
from jax.experimental import pallas as pl
from jax.experimental.pallas import tpu as pltpu
from jax.experimental.pallas import tpu_sc as plsc

B, T = 4096, 50
NTOK = B * T
FINAL = 192

NW = 32
BPW = NTOK // NW
CHUNK = 640
NCHUNK = BPW // CHUNK
SUB = 128
NSUB = CHUNK // SUB

BB = 1024
NB = B // BB


def _sc_gather_rows(table2, idx):
    mesh = plsc.VectorSubcoreMesh(core_axis_name="c", subcore_axis_name="s")

    @functools.partial(
        pl.kernel,
        mesh=mesh,
        out_type=jax.ShapeDtypeStruct((NTOK, 128), jnp.float32),
        scratch_types=[
            pltpu.VMEM((CHUNK,), jnp.int32),
            pltpu.VMEM((CHUNK, 128), jnp.float32),
            pltpu.SemaphoreType.DMA,
        ],
    )
    def gather_kernel(table_hbm, idx_hbm, out_hbm, idx_v, rows_v, sem):
        wid = lax.axis_index("s") * 2 + lax.axis_index("c")
        base0 = wid * BPW
        for j in range(NCHUNK):
            base = base0 + j * CHUNK
            pltpu.sync_copy(idx_hbm.at[pl.ds(base, CHUNK)], idx_v)
            copies = [
                pltpu.async_copy(
                    table_hbm.at[idx_v.at[pl.ds(k * SUB, SUB)]],
                    rows_v.at[pl.ds(k * SUB, SUB)],
                    sem,
                )
                for k in range(NSUB)
            ]
            for c in copies:
                c.wait()
            pltpu.sync_copy(rows_v, out_hbm.at[pl.ds(base, CHUNK)])

    return gather_kernel(table2, idx)


NLOC = 1000000
TBLK = 4096
TGRID = (NLOC + TBLK - 1) // TBLK


def _tpad_body(wt_ref, out_ref):
    eye = (lax.broadcasted_iota(jnp.int32, (64, 128), 0) ==
           lax.broadcasted_iota(jnp.int32, (64, 128), 1)).astype(jnp.float32)
    out_ref[...] = lax.dot_general(
        wt_ref[...], eye, (((0,), (0,)), ((), ())),
        preferred_element_type=jnp.float32)


def _transpose_pad(W_loc):
    wt = lax.transpose(W_loc, (1, 0))
    return pl.pallas_call(
        _tpad_body,
        grid=(TGRID,),
        in_specs=[pl.BlockSpec((64, TBLK), lambda i: (0, i))],
        out_specs=pl.BlockSpec((TBLK, 128), lambda i: (i, 0)),
        out_shape=jax.ShapeDtypeStruct((NLOC, 128), jnp.float32),
    )(wt)


def _tc_body(day_ref, time_ref, dow_ref, wd_ref, locidx_ref, loc3_ref,
             wct_ref, g_ref, b_ref, out_ref):
    t = pl.program_id(1)
    d = day_ref[pl.ds(t, 1), :]
    tt = time_ref[pl.ds(t, 1), :] + 75
    dw = dow_ref[pl.ds(t, 1), :] + 123
    wd = wd_ref[pl.ds(t, 1), :] + 130
    iot = lax.broadcasted_iota(jnp.int32, (132, BB), 0)
    oh = ((iot == d) | (iot == tt) | (iot == dw) | (iot == wd)).astype(
        jnp.bfloat16)
    wct = wct_ref[...]
    wct_hi = wct.astype(jnp.bfloat16)
    wct_lo = (wct - wct_hi.astype(jnp.float32)).astype(jnp.bfloat16)
    small = (jnp.dot(wct_hi, oh, preferred_element_type=jnp.float32) +
             jnp.dot(wct_lo, oh, preferred_element_type=jnp.float32))

    locrows = loc3_ref[0]
    locsel = lax.transpose(locrows, (1, 0))[0:64, :]

    h = jnp.concatenate([small, locsel], axis=0)
    mean = jnp.mean(h, axis=0, keepdims=True)
    hc = h - mean
    var = jnp.mean(hc * hc, axis=0, keepdims=True)
    inv = lax.rsqrt(var + 1e-5)
    out_ref[0] = hc * inv * g_ref[...] + b_ref[...]


def kernel(day, time, dow, weekday, location, W_day, W_time, W_dow,
           W_weekday, W_loc, ln_gamma, ln_beta):
    table2 = _transpose_pad(W_loc)

    day_t = lax.transpose(day.astype(jnp.int32), (1, 0))
    time_t = lax.transpose(time.astype(jnp.int32), (1, 0))
    dow_t = lax.transpose(dow.astype(jnp.int32), (1, 0))
    wd_t = lax.transpose(weekday.astype(jnp.int32), (1, 0))
    loc_t = lax.transpose(location.astype(jnp.int32), (1, 0))

    loc_flat = loc_t.reshape(NTOK)
    loc3 = _sc_gather_rows(table2, loc_flat).reshape(T, B, 128)

    wct = jnp.zeros((128, 132), jnp.float32)
    wct = wct.at[0:32, 0:75].set(W_day.T)
    wct = wct.at[32:64, 75:123].set(W_time.T)
    wct = wct.at[64:96, 123:130].set(W_dow.T)
    wct = wct.at[96:128, 130:132].set(W_weekday.T)

    out_t = pl.pallas_call(
        _tc_body,
        grid=(NB, T),
        in_specs=[
            pl.BlockSpec((T, BB), lambda i, t: (0, i)),
            pl.BlockSpec((T, BB), lambda i, t: (0, i)),
            pl.BlockSpec((T, BB), lambda i, t: (0, i)),
            pl.BlockSpec((T, BB), lambda i, t: (0, i)),
            pl.BlockSpec((T, BB), lambda i, t: (0, i)),
            pl.BlockSpec((1, BB, 128), lambda i, t: (t, i, 0)),
            pl.BlockSpec((128, 132), lambda i, t: (0, 0)),
            pl.BlockSpec((FINAL, 1), lambda i, t: (0, 0)),
            pl.BlockSpec((FINAL, 1), lambda i, t: (0, 0)),
        ],
        out_specs=pl.BlockSpec((1, FINAL, BB), lambda i, t: (t, 0, i)),
        out_shape=jax.ShapeDtypeStruct((T, FINAL, B), jnp.float32),
    )(day_t, time_t, dow_t, wd_t, loc_t, loc3, wct,
      ln_gamma.reshape(FINAL, 1), ln_beta.reshape(FINAL, 1))

    return lax.transpose(out_t, (2, 0, 1))


# speedup vs baseline: 7.2448x; 1.1027x over previous
import functools

import jax
import jax.numpy as jnp
from jax import lax
from jax.experimental import pallas as pl
from jax.experimental.pallas import tpu as pltpu
from jax.experimental.pallas import tpu_sc as plsc

B, T = 4096, 50
NTOK = B * T
FINAL = 192

NW = 32
BPW = NTOK // NW
CHUNK = 640
NCHUNK = BPW // CHUNK
SUB = 128
NSUB = CHUNK // SUB

BB = 2048
NB = B // BB


def _sc_gather_rows(table2, idx):
    mesh = plsc.VectorSubcoreMesh(core_axis_name="c", subcore_axis_name="s")

    @functools.partial(
        pl.kernel,
        mesh=mesh,
        out_type=jax.ShapeDtypeStruct((NTOK, 128), jnp.float32),
        scratch_types=[
            pltpu.VMEM((CHUNK,), jnp.int32),
            pltpu.VMEM((CHUNK, 128), jnp.float32),
            pltpu.SemaphoreType.DMA,
        ],
    )
    def gather_kernel(table_hbm, idx_hbm, out_hbm, idx_v, rows_v, sem):
        wid = lax.axis_index("s") * 2 + lax.axis_index("c")
        base0 = wid * BPW
        for j in range(NCHUNK):
            base = base0 + j * CHUNK
            pltpu.sync_copy(idx_hbm.at[pl.ds(base, CHUNK)], idx_v)
            copies = [
                pltpu.async_copy(
                    table_hbm.at[idx_v.at[pl.ds(k * SUB, SUB)]],
                    rows_v.at[pl.ds(k * SUB, SUB)],
                    sem,
                )
                for k in range(NSUB)
            ]
            for c in copies:
                c.wait()
            pltpu.sync_copy(rows_v, out_hbm.at[pl.ds(base, CHUNK)])

    return gather_kernel(table2, idx)


NLOC = 1000000
TBLK = 4096
TGRID = (NLOC + TBLK - 1) // TBLK


def _tpad_body(wt_ref, out_ref):
    tr = lax.transpose(wt_ref[...], (1, 0))
    out_ref[...] = jnp.concatenate(
        [tr, jnp.zeros((TBLK, 64), jnp.float32)], axis=1)


def _transpose_pad(W_loc):
    wt = lax.transpose(W_loc, (1, 0))
    return pl.pallas_call(
        _tpad_body,
        grid=(TGRID,),
        in_specs=[pl.BlockSpec((64, TBLK), lambda i: (0, i))],
        out_specs=pl.BlockSpec((TBLK, 128), lambda i: (i, 0)),
        out_shape=jax.ShapeDtypeStruct((NLOC, 128), jnp.float32),
    )(wt)


def _tc_body(day_ref, time_ref, dow_ref, wd_ref, locidx_ref, loc3_ref,
             wct_ref, g_ref, b_ref, out_ref):
    t = pl.program_id(1)
    d = day_ref[pl.ds(t, 1), :]
    tt = time_ref[pl.ds(t, 1), :] + 75
    dw = dow_ref[pl.ds(t, 1), :] + 123
    wd = wd_ref[pl.ds(t, 1), :] + 130
    iot = lax.broadcasted_iota(jnp.int32, (132, BB), 0)
    oh = ((iot == d) | (iot == tt) | (iot == dw) | (iot == wd)).astype(
        jnp.bfloat16)
    wct = wct_ref[...]
    wct_hi = wct.astype(jnp.bfloat16)
    wct_lo = (wct - wct_hi.astype(jnp.float32)).astype(jnp.bfloat16)
    small = (jnp.dot(wct_hi, oh, preferred_element_type=jnp.float32) +
             jnp.dot(wct_lo, oh, preferred_element_type=jnp.float32))

    locrows = loc3_ref[0, :, 0:64]
    locsel = lax.transpose(locrows, (1, 0))

    h = jnp.concatenate([small, locsel], axis=0)
    mean = jnp.mean(h, axis=0, keepdims=True)
    hc = h - mean
    var = jnp.mean(hc * hc, axis=0, keepdims=True)
    inv = lax.rsqrt(var + 1e-5)
    out_ref[0] = hc * inv * g_ref[...] + b_ref[...]


def kernel(day, time, dow, weekday, location, W_day, W_time, W_dow,
           W_weekday, W_loc, ln_gamma, ln_beta):
    table2 = _transpose_pad(W_loc)

    day_t = lax.transpose(day.astype(jnp.int32), (1, 0))
    time_t = lax.transpose(time.astype(jnp.int32), (1, 0))
    dow_t = lax.transpose(dow.astype(jnp.int32), (1, 0))
    wd_t = lax.transpose(weekday.astype(jnp.int32), (1, 0))
    loc_t = lax.transpose(location.astype(jnp.int32), (1, 0))

    loc_flat = loc_t.reshape(NTOK)
    loc3 = _sc_gather_rows(table2, loc_flat).reshape(T, B, 128)

    wct = jnp.zeros((128, 132), jnp.float32)
    wct = wct.at[0:32, 0:75].set(W_day.T)
    wct = wct.at[32:64, 75:123].set(W_time.T)
    wct = wct.at[64:96, 123:130].set(W_dow.T)
    wct = wct.at[96:128, 130:132].set(W_weekday.T)

    out_t = pl.pallas_call(
        _tc_body,
        grid=(NB, T),
        in_specs=[
            pl.BlockSpec((T, BB), lambda i, t: (0, i)),
            pl.BlockSpec((T, BB), lambda i, t: (0, i)),
            pl.BlockSpec((T, BB), lambda i, t: (0, i)),
            pl.BlockSpec((T, BB), lambda i, t: (0, i)),
            pl.BlockSpec((T, BB), lambda i, t: (0, i)),
            pl.BlockSpec((1, BB, 128), lambda i, t: (t, i, 0)),
            pl.BlockSpec((128, 132), lambda i, t: (0, 0)),
            pl.BlockSpec((FINAL, 1), lambda i, t: (0, 0)),
            pl.BlockSpec((FINAL, 1), lambda i, t: (0, 0)),
        ],
        out_specs=pl.BlockSpec((1, FINAL, BB), lambda i, t: (t, 0, i)),
        out_shape=jax.ShapeDtypeStruct((T, FINAL, B), jnp.float32),
    )(day_t, time_t, dow_t, wd_t, loc_t, loc3, wct,
      ln_gamma.reshape(FINAL, 1), ln_beta.reshape(FINAL, 1))

    return lax.transpose(out_t, (2, 0, 1))


# speedup vs baseline: 7.3308x; 1.0119x over previous
import functools

import jax
import jax.numpy as jnp
from jax import lax
from jax.experimental import pallas as pl
from jax.experimental.pallas import tpu as pltpu
from jax.experimental.pallas import tpu_sc as plsc

B, T = 4096, 50
NTOK = B * T
FINAL = 192

NW = 32
BPW = NTOK // NW
CHUNK = 256
NCHUNK = BPW // CHUNK
SUB = 128
NSUB = CHUNK // SUB

BB = 2048
NB = B // BB


def _sc_gather_rows(table2, idx):
    mesh = plsc.VectorSubcoreMesh(core_axis_name="c", subcore_axis_name="s")

    @functools.partial(
        pl.kernel,
        mesh=mesh,
        out_type=jax.ShapeDtypeStruct((NTOK, 128), jnp.float32),
        scratch_types=[
            pltpu.VMEM((BPW,), jnp.int32),
            pltpu.VMEM((CHUNK, 128), jnp.float32),
            pltpu.VMEM((CHUNK, 128), jnp.float32),
            pltpu.SemaphoreType.DMA,
            pltpu.SemaphoreType.DMA,
            pltpu.SemaphoreType.DMA,
            pltpu.SemaphoreType.DMA,
        ],
    )
    def gather_kernel(table_hbm, idx_hbm, out_hbm, idx_v, rows0, rows1,
                      g0, g1, w0, w1, ):
        wid = lax.axis_index("s") * 2 + lax.axis_index("c")
        base0 = wid * BPW
        rows = (rows0, rows1)
        gsem = (g0, g1)
        wsem = (w0, w1)
        pltpu.sync_copy(idx_hbm.at[pl.ds(base0, BPW)], idx_v)

        def fire(j, b):
            return [
                pltpu.async_copy(
                    table_hbm.at[idx_v.at[pl.ds(j * CHUNK + k * SUB, SUB)]],
                    rows[b].at[pl.ds(k * SUB, SUB)],
                    gsem[b],
                )
                for k in range(NSUB)
            ]

        ga = [fire(0, 0), None]
        wb = [None, None]
        for j in range(NCHUNK):
            b = j % 2
            nb = (j + 1) % 2
            if j + 1 < NCHUNK:
                if wb[nb] is not None:
                    wb[nb].wait()
                    wb[nb] = None
                ga[nb] = fire(j + 1, nb)
            for c in ga[b]:
                c.wait()
            if wb[b] is not None:
                wb[b].wait()
            wb[b] = pltpu.async_copy(
                rows[b], out_hbm.at[pl.ds(base0 + j * CHUNK, CHUNK)],
                wsem[b])
        for b in (0, 1):
            if wb[b] is not None:
                wb[b].wait()

    return gather_kernel(table2, idx)


NLOC = 1000000
TBLK = 4096
TGRID = (NLOC + TBLK - 1) // TBLK


def _tpad_body(wt_ref, out_ref):
    tr = lax.transpose(wt_ref[...], (1, 0))
    out_ref[...] = jnp.concatenate(
        [tr, jnp.zeros((TBLK, 64), jnp.float32)], axis=1)


def _transpose_pad(W_loc):
    wt = lax.transpose(W_loc, (1, 0))
    return pl.pallas_call(
        _tpad_body,
        grid=(TGRID,),
        in_specs=[pl.BlockSpec((64, TBLK), lambda i: (0, i))],
        out_specs=pl.BlockSpec((TBLK, 128), lambda i: (i, 0)),
        out_shape=jax.ShapeDtypeStruct((NLOC, 128), jnp.float32),
    )(wt)


def _tc_body(day_ref, time_ref, dow_ref, wd_ref, locidx_ref, loc3_ref,
             wct_ref, g_ref, b_ref, out_ref):
    t = pl.program_id(1)
    d = day_ref[pl.ds(t, 1), :]
    tt = time_ref[pl.ds(t, 1), :] + 75
    dw = dow_ref[pl.ds(t, 1), :] + 123
    wd = wd_ref[pl.ds(t, 1), :] + 130
    iot = lax.broadcasted_iota(jnp.int32, (132, BB), 0)
    oh = ((iot == d) | (iot == tt) | (iot == dw) | (iot == wd)).astype(
        jnp.bfloat16)
    wct = wct_ref[...]
    wct_hi = wct.astype(jnp.bfloat16)
    wct_lo = (wct - wct_hi.astype(jnp.float32)).astype(jnp.bfloat16)
    small = (jnp.dot(wct_hi, oh, preferred_element_type=jnp.float32) +
             jnp.dot(wct_lo, oh, preferred_element_type=jnp.float32))

    locrows = loc3_ref[0, :, 0:64]
    locsel = lax.transpose(locrows, (1, 0))

    h = jnp.concatenate([small, locsel], axis=0)
    mean = jnp.mean(h, axis=0, keepdims=True)
    hc = h - mean
    var = jnp.mean(hc * hc, axis=0, keepdims=True)
    inv = lax.rsqrt(var + 1e-5)
    out_ref[0] = hc * inv * g_ref[...] + b_ref[...]


def kernel(day, time, dow, weekday, location, W_day, W_time, W_dow,
           W_weekday, W_loc, ln_gamma, ln_beta):
    table2 = _transpose_pad(W_loc)

    day_t = lax.transpose(day.astype(jnp.int32), (1, 0))
    time_t = lax.transpose(time.astype(jnp.int32), (1, 0))
    dow_t = lax.transpose(dow.astype(jnp.int32), (1, 0))
    wd_t = lax.transpose(weekday.astype(jnp.int32), (1, 0))
    loc_t = lax.transpose(location.astype(jnp.int32), (1, 0))

    loc_flat = loc_t.reshape(NTOK)
    loc3 = _sc_gather_rows(table2, loc_flat).reshape(T, B, 128)

    wct = jnp.zeros((128, 132), jnp.float32)
    wct = wct.at[0:32, 0:75].set(W_day.T)
    wct = wct.at[32:64, 75:123].set(W_time.T)
    wct = wct.at[64:96, 123:130].set(W_dow.T)
    wct = wct.at[96:128, 130:132].set(W_weekday.T)

    out_t = pl.pallas_call(
        _tc_body,
        grid=(NB, T),
        in_specs=[
            pl.BlockSpec((T, BB), lambda i, t: (0, i)),
            pl.BlockSpec((T, BB), lambda i, t: (0, i)),
            pl.BlockSpec((T, BB), lambda i, t: (0, i)),
            pl.BlockSpec((T, BB), lambda i, t: (0, i)),
            pl.BlockSpec((T, BB), lambda i, t: (0, i)),
            pl.BlockSpec((1, BB, 128), lambda i, t: (t, i, 0)),
            pl.BlockSpec((128, 132), lambda i, t: (0, 0)),
            pl.BlockSpec((FINAL, 1), lambda i, t: (0, 0)),
            pl.BlockSpec((FINAL, 1), lambda i, t: (0, 0)),
        ],
        out_specs=pl.BlockSpec((1, FINAL, BB), lambda i, t: (t, 0, i)),
        out_shape=jax.ShapeDtypeStruct((T, FINAL, B), jnp.float32),
    )(day_t, time_t, dow_t, wd_t, loc_t, loc3, wct,
      ln_gamma.reshape(FINAL, 1), ln_beta.reshape(FINAL, 1))

    return lax.transpose(out_t, (2, 0, 1))


# speedup vs baseline: 8.3671x; 1.1414x over previous
import functools

import jax
import jax.numpy as jnp
from jax import lax
from jax.experimental import pallas as pl
from jax.experimental.pallas import tpu as pltpu
from jax.experimental.pallas import tpu_sc as plsc

B, T = 4096, 50
NTOK = B * T
FINAL = 192

NW = 32
BPW = NTOK // NW
CHUNK = 256
NCHUNK = BPW // CHUNK
SUB = 128
NSUB = CHUNK // SUB

BB = 2048
NB = B // BB


def _sc_gather_rows(table2, idx):
    mesh = plsc.VectorSubcoreMesh(core_axis_name="c", subcore_axis_name="s")

    @functools.partial(
        pl.kernel,
        mesh=mesh,
        out_type=jax.ShapeDtypeStruct((NTOK, 128), jnp.float32),
        scratch_types=[
            pltpu.VMEM((BPW,), jnp.int32),
            pltpu.VMEM((CHUNK, 128), jnp.float32),
            pltpu.VMEM((CHUNK, 128), jnp.float32),
            pltpu.SemaphoreType.DMA,
            pltpu.SemaphoreType.DMA,
            pltpu.SemaphoreType.DMA,
            pltpu.SemaphoreType.DMA,
        ],
    )
    def gather_kernel(table_hbm, idx_hbm, out_hbm, idx_v, rows0, rows1,
                      g0, g1, w0, w1, ):
        wid = lax.axis_index("s") * 2 + lax.axis_index("c")
        base0 = wid * BPW
        rows = (rows0, rows1)
        gsem = (g0, g1)
        wsem = (w0, w1)
        pltpu.sync_copy(idx_hbm.at[pl.ds(base0, BPW)], idx_v)

        def fire(j, b):
            return [
                pltpu.async_copy(
                    table_hbm.at[idx_v.at[pl.ds(j * CHUNK + k * SUB, SUB)]],
                    rows[b].at[pl.ds(k * SUB, SUB)],
                    gsem[b],
                )
                for k in range(NSUB)
            ]

        ga = [fire(0, 0), None]
        wb = [None, None]
        for j in range(NCHUNK):
            b = j % 2
            nb = (j + 1) % 2
            if j + 1 < NCHUNK:
                if wb[nb] is not None:
                    wb[nb].wait()
                    wb[nb] = None
                ga[nb] = fire(j + 1, nb)
            for c in ga[b]:
                c.wait()
            if wb[b] is not None:
                wb[b].wait()
            wb[b] = pltpu.async_copy(
                rows[b], out_hbm.at[pl.ds(base0 + j * CHUNK, CHUNK)],
                wsem[b])
        for b in (0, 1):
            if wb[b] is not None:
                wb[b].wait()

    return gather_kernel(table2, idx)


NLOC = 1000000
TBLK = 8192
TGRID = (NLOC + TBLK - 1) // TBLK


def _tpad_body(wt_ref, out_ref):
    tr = lax.transpose(wt_ref[...], (1, 0))
    out_ref[...] = jnp.concatenate(
        [tr, jnp.zeros((TBLK, 64), jnp.float32)], axis=1)


def _transpose_pad(W_loc):
    wt = lax.transpose(W_loc, (1, 0))
    return pl.pallas_call(
        _tpad_body,
        grid=(TGRID,),
        in_specs=[pl.BlockSpec((64, TBLK), lambda i: (0, i))],
        out_specs=pl.BlockSpec((TBLK, 128), lambda i: (i, 0)),
        out_shape=jax.ShapeDtypeStruct((NLOC, 128), jnp.float32),
    )(wt)


def _tc_body(day_ref, time_ref, dow_ref, wd_ref, locidx_ref, loc3_ref,
             wct_ref, g_ref, b_ref, out_ref):
    t = pl.program_id(1)
    d = day_ref[pl.ds(t, 1), :]
    tt = time_ref[pl.ds(t, 1), :] + 75
    dw = dow_ref[pl.ds(t, 1), :] + 123
    wd = wd_ref[pl.ds(t, 1), :] + 130
    iot = lax.broadcasted_iota(jnp.int32, (132, BB), 0)
    oh = ((iot == d) | (iot == tt) | (iot == dw) | (iot == wd)).astype(
        jnp.bfloat16)
    wct = wct_ref[...]
    wct_hi = wct.astype(jnp.bfloat16)
    wct_lo = (wct - wct_hi.astype(jnp.float32)).astype(jnp.bfloat16)
    small = (jnp.dot(wct_hi, oh, preferred_element_type=jnp.float32) +
             jnp.dot(wct_lo, oh, preferred_element_type=jnp.float32))

    locrows = loc3_ref[0, :, 0:64]
    locsel = lax.transpose(locrows, (1, 0))

    h = jnp.concatenate([small, locsel], axis=0)
    mean = jnp.mean(h, axis=0, keepdims=True)
    hc = h - mean
    var = jnp.mean(hc * hc, axis=0, keepdims=True)
    inv = lax.rsqrt(var + 1e-5)
    out_ref[0] = hc * inv * g_ref[...] + b_ref[...]


def kernel(day, time, dow, weekday, location, W_day, W_time, W_dow,
           W_weekday, W_loc, ln_gamma, ln_beta):
    table2 = _transpose_pad(W_loc)

    day_t = lax.transpose(day.astype(jnp.int32), (1, 0))
    time_t = lax.transpose(time.astype(jnp.int32), (1, 0))
    dow_t = lax.transpose(dow.astype(jnp.int32), (1, 0))
    wd_t = lax.transpose(weekday.astype(jnp.int32), (1, 0))
    loc_t = lax.transpose(location.astype(jnp.int32), (1, 0))

    loc_flat = loc_t.reshape(NTOK)
    loc3 = _sc_gather_rows(table2, loc_flat).reshape(T, B, 128)

    wct = jnp.zeros((128, 132), jnp.float32)
    wct = wct.at[0:32, 0:75].set(W_day.T)
    wct = wct.at[32:64, 75:123].set(W_time.T)
    wct = wct.at[64:96, 123:130].set(W_dow.T)
    wct = wct.at[96:128, 130:132].set(W_weekday.T)

    out_t = pl.pallas_call(
        _tc_body,
        grid=(NB, T),
        in_specs=[
            pl.BlockSpec((T, BB), lambda i, t: (0, i)),
            pl.BlockSpec((T, BB), lambda i, t: (0, i)),
            pl.BlockSpec((T, BB), lambda i, t: (0, i)),
            pl.BlockSpec((T, BB), lambda i, t: (0, i)),
            pl.BlockSpec((T, BB), lambda i, t: (0, i)),
            pl.BlockSpec((1, BB, 128), lambda i, t: (t, i, 0)),
            pl.BlockSpec((128, 132), lambda i, t: (0, 0)),
            pl.BlockSpec((FINAL, 1), lambda i, t: (0, 0)),
            pl.BlockSpec((FINAL, 1), lambda i, t: (0, 0)),
        ],
        out_specs=pl.BlockSpec((1, FINAL, BB), lambda i, t: (t, 0, i)),
        out_shape=jax.ShapeDtypeStruct((T, FINAL, B), jnp.float32),
    )(day_t, time_t, dow_t, wd_t, loc_t, loc3, wct,
      ln_gamma.reshape(FINAL, 1), ln_beta.reshape(FINAL, 1))

    return lax.transpose(out_t, (2, 0, 1))


# speedup vs baseline: 8.6769x; 1.0370x over previous
import functools

import jax
import jax.numpy as jnp
from jax import lax
from jax.experimental import pallas as pl
from jax.experimental.pallas import tpu as pltpu
from jax.experimental.pallas import tpu_sc as plsc

B, T = 4096, 50
NTOK = B * T
FINAL = 192

NW = 32
BPW = NTOK // NW
CHUNK = 256
NCHUNK = BPW // CHUNK
SUB = 128
NSUB = CHUNK // SUB

BB = 2048
NB = B // BB


def _sc_gather_rows(table2, idx):
    mesh = plsc.VectorSubcoreMesh(core_axis_name="c", subcore_axis_name="s")

    @functools.partial(
        pl.kernel,
        mesh=mesh,
        out_type=jax.ShapeDtypeStruct((NTOK, 128), jnp.float32),
        scratch_types=[
            pltpu.VMEM((BPW,), jnp.int32),
            pltpu.VMEM((CHUNK, 128), jnp.float32),
            pltpu.VMEM((CHUNK, 128), jnp.float32),
            pltpu.SemaphoreType.DMA,
            pltpu.SemaphoreType.DMA,
            pltpu.SemaphoreType.DMA,
            pltpu.SemaphoreType.DMA,
        ],
    )
    def gather_kernel(table_hbm, idx_hbm, out_hbm, idx_v, rows0, rows1,
                      g0, g1, w0, w1, ):
        wid = lax.axis_index("s") * 2 + lax.axis_index("c")
        base0 = wid * BPW
        rows = (rows0, rows1)
        gsem = (g0, g1)
        wsem = (w0, w1)
        pltpu.sync_copy(idx_hbm.at[pl.ds(base0, BPW)], idx_v)

        def fire(j, b):
            return [
                pltpu.async_copy(
                    table_hbm.at[idx_v.at[pl.ds(j * CHUNK + k * SUB, SUB)]],
                    rows[b].at[pl.ds(k * SUB, SUB)],
                    gsem[b],
                )
                for k in range(NSUB)
            ]

        ga = [fire(0, 0), None]
        wb = [None, None]
        for j in range(NCHUNK):
            b = j % 2
            nb = (j + 1) % 2
            if j + 1 < NCHUNK:
                if wb[nb] is not None:
                    wb[nb].wait()
                    wb[nb] = None
                ga[nb] = fire(j + 1, nb)
            for c in ga[b]:
                c.wait()
            if wb[b] is not None:
                wb[b].wait()
            wb[b] = pltpu.async_copy(
                rows[b], out_hbm.at[pl.ds(base0 + j * CHUNK, CHUNK)],
                wsem[b])
        for b in (0, 1):
            if wb[b] is not None:
                wb[b].wait()

    return gather_kernel(table2, idx)


NLOC = 1000000
TBLK = 16384
TGRID = (NLOC + TBLK - 1) // TBLK


def _tpad_body(wt_ref, out_ref):
    tr = lax.transpose(wt_ref[...], (1, 0))
    out_ref[...] = jnp.concatenate(
        [tr, jnp.zeros((TBLK, 64), jnp.float32)], axis=1)


def _transpose_pad(W_loc):
    wt = lax.transpose(W_loc, (1, 0))
    return pl.pallas_call(
        _tpad_body,
        grid=(TGRID,),
        in_specs=[pl.BlockSpec((64, TBLK), lambda i: (0, i))],
        out_specs=pl.BlockSpec((TBLK, 128), lambda i: (i, 0)),
        out_shape=jax.ShapeDtypeStruct((NLOC, 128), jnp.float32),
    )(wt)


def _tc_body(day_ref, time_ref, dow_ref, wd_ref, locidx_ref, loc3_ref,
             wct_ref, g_ref, b_ref, out_ref):
    t = pl.program_id(1)
    d = day_ref[pl.ds(t, 1), :]
    tt = time_ref[pl.ds(t, 1), :] + 75
    dw = dow_ref[pl.ds(t, 1), :] + 123
    wd = wd_ref[pl.ds(t, 1), :] + 130
    iot = lax.broadcasted_iota(jnp.int32, (132, BB), 0)
    oh = ((iot == d) | (iot == tt) | (iot == dw) | (iot == wd)).astype(
        jnp.bfloat16)
    wct = wct_ref[...]
    wct_hi = wct.astype(jnp.bfloat16)
    wct_lo = (wct - wct_hi.astype(jnp.float32)).astype(jnp.bfloat16)
    small = (jnp.dot(wct_hi, oh, preferred_element_type=jnp.float32) +
             jnp.dot(wct_lo, oh, preferred_element_type=jnp.float32))

    locrows = loc3_ref[0, :, 0:64]
    locsel = lax.transpose(locrows, (1, 0))

    h = jnp.concatenate([small, locsel], axis=0)
    mean = jnp.mean(h, axis=0, keepdims=True)
    hc = h - mean
    var = jnp.mean(hc * hc, axis=0, keepdims=True)
    inv = lax.rsqrt(var + 1e-5)
    out_ref[0] = hc * inv * g_ref[...] + b_ref[...]


def kernel(day, time, dow, weekday, location, W_day, W_time, W_dow,
           W_weekday, W_loc, ln_gamma, ln_beta):
    table2 = _transpose_pad(W_loc)

    day_t = lax.transpose(day.astype(jnp.int32), (1, 0))
    time_t = lax.transpose(time.astype(jnp.int32), (1, 0))
    dow_t = lax.transpose(dow.astype(jnp.int32), (1, 0))
    wd_t = lax.transpose(weekday.astype(jnp.int32), (1, 0))
    loc_t = lax.transpose(location.astype(jnp.int32), (1, 0))

    loc_flat = loc_t.reshape(NTOK)
    loc3 = _sc_gather_rows(table2, loc_flat).reshape(T, B, 128)

    wct = jnp.zeros((128, 132), jnp.float32)
    wct = wct.at[0:32, 0:75].set(W_day.T)
    wct = wct.at[32:64, 75:123].set(W_time.T)
    wct = wct.at[64:96, 123:130].set(W_dow.T)
    wct = wct.at[96:128, 130:132].set(W_weekday.T)

    out_t = pl.pallas_call(
        _tc_body,
        grid=(NB, T),
        in_specs=[
            pl.BlockSpec((T, BB), lambda i, t: (0, i)),
            pl.BlockSpec((T, BB), lambda i, t: (0, i)),
            pl.BlockSpec((T, BB), lambda i, t: (0, i)),
            pl.BlockSpec((T, BB), lambda i, t: (0, i)),
            pl.BlockSpec((T, BB), lambda i, t: (0, i)),
            pl.BlockSpec((1, BB, 128), lambda i, t: (t, i, 0)),
            pl.BlockSpec((128, 132), lambda i, t: (0, 0)),
            pl.BlockSpec((FINAL, 1), lambda i, t: (0, 0)),
            pl.BlockSpec((FINAL, 1), lambda i, t: (0, 0)),
        ],
        out_specs=pl.BlockSpec((1, FINAL, BB), lambda i, t: (t, 0, i)),
        out_shape=jax.ShapeDtypeStruct((T, FINAL, B), jnp.float32),
    )(day_t, time_t, dow_t, wd_t, loc_t, loc3, wct,
      ln_gamma.reshape(FINAL, 1), ln_beta.reshape(FINAL, 1))

    return lax.transpose(out_t, (2, 0, 1))
